# baseline (device time: 16233 ns/iter reference)
import jax
import jax.numpy as jnp
from jax import lax
from jax.experimental import pallas as pl
from jax.experimental.pallas import tpu as pltpu

R = 32
N_F = 13
N_D = 6
N_X = N_F + N_D
F_ROWS = R * N_F
D_BASE = 2 * F_ROWS

F_GROUPS = [(0, 4), (4, 8), (8, 12), (12, 13)]
D_GROUPS = [(13, 17), (17, 19)]
SCHEDULE = [
    ("F", F_GROUPS[0]), ("F", F_GROUPS[1]), ("F", F_GROUPS[2]),
    ("Y", F_GROUPS[0]), ("F", F_GROUPS[3]), ("Y", F_GROUPS[1]),
    ("D", D_GROUPS[0]), ("Y", F_GROUPS[2]), ("D", D_GROUPS[1]),
    ("Y", F_GROUPS[3]),
]


def kernel(x):
    m_per, n = x.shape
    m_glob = 2 * m_per
    n_per = n // 2
    assert 2 * F_ROWS + R * N_D == m_per

    def body(x_ref, out_ref, send_buf, rx_buf, ry_buf, sx, rx_sem, sy, ry_sem):
        mx = lax.axis_index("x")
        my = lax.axis_index("y")
        px = 1 - mx
        py = 1 - my

        barrier_sem = pltpu.get_barrier_semaphore()
        pl.semaphore_signal(
            barrier_sem, inc=1,
            device_id=(px, my), device_id_type=pl.DeviceIdType.MESH,
        )
        pl.semaphore_signal(
            barrier_sem, inc=1,
            device_id=(mx, py), device_id_type=pl.DeviceIdType.MESH,
        )
        pl.semaphore_wait(barrier_sem, 2)

        def x_rdma(i):
            return pltpu.make_async_remote_copy(
                src_ref=send_buf.at[pl.ds(i * R, R), :],
                dst_ref=rx_buf.at[pl.ds(i * R, R), :],
                send_sem=sx.at[i],
                recv_sem=rx_sem.at[i],
                device_id=(px, my),
                device_id_type=pl.DeviceIdType.MESH,
            )

        def y_rdma(i):
            return pltpu.make_async_remote_copy(
                src_ref=rx_buf.at[pl.ds(i * R, R), :],
                dst_ref=ry_buf.at[pl.ds(i * R, R), :],
                send_sem=sy.at[i],
                recv_sem=ry_sem.at[i],
                device_id=(mx, py),
                device_id_type=pl.DeviceIdType.MESH,
            )

        def src_row(s):
            if s < N_F:
                return my * F_ROWS + s * R
            return D_BASE + (s - N_F) * R

        for s, e in F_GROUPS + D_GROUPS:
            rr = (e - s) * R
            send_buf[pl.ds(s * R, rr), :] = x_ref[
                pl.ds(src_row(s), rr), pl.ds(px * n_per, n_per)
            ].astype(jnp.bfloat16)
            for i in range(s, e):
                x_rdma(i).start()

        out_ref[pl.ds(mx * m_per, m_per), :] = x_ref[:, pl.ds(mx * n_per, n_per)]

        for kind, (s, e) in SCHEDULE:
            rr = (e - s) * R
            if kind == "F":
                for i in range(s, e):
                    x_rdma(i).wait_recv()
                    y_rdma(i).start()
                out_ref[pl.ds(px * m_per + my * F_ROWS + s * R, rr), :] = rx_buf[
                    pl.ds(s * R, rr), :
                ].astype(jnp.float32)
            elif kind == "D":
                for i in range(s, e):
                    x_rdma(i).wait_recv()
                out_ref[
                    pl.ds(px * m_per + D_BASE + (s - N_F) * R, rr), :
                ] = rx_buf[pl.ds(s * R, rr), :].astype(jnp.float32)
            else:
                for i in range(s, e):
                    y_rdma(i).wait_recv()
                out_ref[pl.ds(px * m_per + py * F_ROWS + s * R, rr), :] = ry_buf[
                    pl.ds(s * R, rr), :
                ].astype(jnp.float32)

        for i in range(N_X):
            x_rdma(i).wait_send()
        for i in range(N_F):
            y_rdma(i).wait_send()

    return pl.pallas_call(
        body,
        out_shape=jax.ShapeDtypeStruct((m_glob, n_per), x.dtype),
        in_specs=[pl.BlockSpec(memory_space=pltpu.VMEM)],
        out_specs=pl.BlockSpec(memory_space=pltpu.VMEM),
        scratch_shapes=[
            pltpu.VMEM((N_X * R, n_per), jnp.bfloat16),
            pltpu.VMEM((N_X * R, n_per), jnp.bfloat16),
            pltpu.VMEM((N_F * R, n_per), jnp.bfloat16),
            pltpu.SemaphoreType.DMA((N_X,)),
            pltpu.SemaphoreType.DMA((N_X,)),
            pltpu.SemaphoreType.DMA((N_F,)),
            pltpu.SemaphoreType.DMA((N_F,)),
        ],
        compiler_params=pltpu.CompilerParams(collective_id=0),
    )(x)


# device time: 16162 ns/iter; 1.0044x vs baseline; 1.0044x over previous
import jax
import jax.numpy as jnp
from jax import lax
from jax.experimental import pallas as pl
from jax.experimental.pallas import tpu as pltpu

R = 64
N_F = 7
N_D = 2
N_X = N_F + N_D
F_ROWS = R * N_F
D_BASE = 2 * F_ROWS


def kernel(x):
    m_per, n = x.shape
    m_glob = 2 * m_per
    n_per = n // 2
    assert 2 * F_ROWS + R * N_D == m_per

    def body(x_ref, out_ref, send_buf, rx_buf, ry_buf, sx, rx_sem, sy, ry_sem):
        mx = lax.axis_index("x")
        my = lax.axis_index("y")
        px = 1 - mx
        py = 1 - my

        barrier_sem = pltpu.get_barrier_semaphore()
        pl.semaphore_signal(
            barrier_sem, inc=1,
            device_id=(px, my), device_id_type=pl.DeviceIdType.MESH,
        )
        pl.semaphore_signal(
            barrier_sem, inc=1,
            device_id=(mx, py), device_id_type=pl.DeviceIdType.MESH,
        )
        pl.semaphore_wait(barrier_sem, 2)

        def x_rdma(i):
            return pltpu.make_async_remote_copy(
                src_ref=send_buf.at[i],
                dst_ref=rx_buf.at[i],
                send_sem=sx.at[i],
                recv_sem=rx_sem.at[i],
                device_id=(px, my),
                device_id_type=pl.DeviceIdType.MESH,
            )

        def y_rdma(i):
            return pltpu.make_async_remote_copy(
                src_ref=rx_buf.at[i],
                dst_ref=ry_buf.at[i],
                send_sem=sy.at[i],
                recv_sem=ry_sem.at[i],
                device_id=(mx, py),
                device_id_type=pl.DeviceIdType.MESH,
            )

        def src_row(i):
            if i < N_F:
                return my * F_ROWS + i * R
            return D_BASE + (i - N_F) * R

        for i in range(N_X):
            send_buf[i, :, :] = x_ref[
                pl.ds(src_row(i), R), pl.ds(px * n_per, n_per)
            ].astype(jnp.bfloat16)
            x_rdma(i).start()

        out_ref[pl.ds(mx * m_per, m_per), :] = x_ref[:, pl.ds(mx * n_per, n_per)]

        Y_LAG = 3
        y_done = 0

        def drain_y(upto):
            nonlocal y_done
            while y_done < min(upto, N_F):
                i = y_done
                y_rdma(i).wait_recv()
                out_ref[pl.ds(px * m_per + py * F_ROWS + i * R, R), :] = ry_buf[
                    i
                ].astype(jnp.float32)
                y_done += 1

        for i in range(N_F):
            x_rdma(i).wait_recv()
            y_rdma(i).start()
            out_ref[pl.ds(px * m_per + my * F_ROWS + i * R, R), :] = rx_buf[
                i
            ].astype(jnp.float32)
            drain_y(i - Y_LAG + 1)

        for j in range(N_D):
            i = N_F + j
            x_rdma(i).wait_recv()
            out_ref[pl.ds(px * m_per + D_BASE + j * R, R), :] = rx_buf[i].astype(
                jnp.float32
            )
            drain_y(y_done + 1)
        drain_y(N_F)

        for i in range(N_X):
            x_rdma(i).wait_send()
        for i in range(N_F):
            y_rdma(i).wait_send()

    return pl.pallas_call(
        body,
        out_shape=jax.ShapeDtypeStruct((m_glob, n_per), x.dtype),
        in_specs=[pl.BlockSpec(memory_space=pltpu.VMEM)],
        out_specs=pl.BlockSpec(memory_space=pltpu.VMEM),
        scratch_shapes=[
            pltpu.VMEM((N_X, R, n_per), jnp.bfloat16),
            pltpu.VMEM((N_X, R, n_per), jnp.bfloat16),
            pltpu.VMEM((N_F, R, n_per), jnp.bfloat16),
            pltpu.SemaphoreType.DMA((N_X,)),
            pltpu.SemaphoreType.DMA((N_X,)),
            pltpu.SemaphoreType.DMA((N_F,)),
            pltpu.SemaphoreType.DMA((N_F,)),
        ],
        compiler_params=pltpu.CompilerParams(collective_id=0),
    )(x)
